# baseline (speedup 1.0000x reference)
import functools

import jax
import jax.numpy as jnp
from jax import lax
from jax.experimental import pallas as pl
from jax.experimental.pallas import tpu as pltpu
from jax.experimental.pallas import tpu_sc as plsc

TOPK = 16
BB = 8
NEG_INF = float("-inf")
LANES = 16


def _topk_body(lite_ref, idx_ref, *, T):
    scores = jnp.max(lite_ref[...], axis=2)
    t_iota = jax.lax.broadcasted_iota(jnp.int32, scores.shape, 1)
    b_iota = jax.lax.broadcasted_iota(jnp.int32, scores.shape, 0)
    row_base = (pl.program_id(0) * BB + b_iota) * T
    k_iota = jax.lax.broadcasted_iota(jnp.int32, (BB, TOPK), 1)
    ind_rows = jnp.zeros((BB, TOPK), jnp.int32)
    big = jnp.int32(2**30)
    for k in range(TOPK):
        m = jnp.max(scores, axis=1, keepdims=True)
        cand = jnp.where(scores == m, t_iota, big)
        idx = jnp.min(cand, axis=1, keepdims=True)
        ind_rows = jnp.where(k_iota == k, idx + row_base[:, :1], ind_rows)
        scores = jnp.where(t_iota == idx, NEG_INF, scores)
    idx_ref[...] = ind_rows.reshape(BB, 1, TOPK)


def _sc_gather_mean_body(idx_hbm, in_hbm, out_hbm, idx_v, rows_v, out_v, sem):
    nc = 2
    wid = lax.axis_index("s") * nc + lax.axis_index("c")
    rows_per_w = 2 * TOPK
    pltpu.sync_copy(idx_hbm.at[pl.ds(wid * rows_per_w, rows_per_w)], idx_v)
    pltpu.async_copy(in_hbm.at[idx_v], rows_v, sem).wait()
    D = in_hbm.shape[1]
    for bb in range(2):
        def c_body(c, carry):
            sl = pl.ds(c * LANES, LANES)
            acc = rows_v[bb * TOPK, sl]
            for r in range(1, TOPK):
                acc = acc + rows_v[bb * TOPK + r, sl]
            out_v[bb, sl] = acc * (1.0 / TOPK)
            return carry

        lax.fori_loop(0, D // LANES, c_body, 0)
    pltpu.sync_copy(out_v, out_hbm.at[pl.ds(wid * 2, 2)])


@jax.jit
def kernel(input, lite_input):
    B, T, D = input.shape

    indices = pl.pallas_call(
        functools.partial(_topk_body, T=T),
        grid=(B // BB,),
        in_specs=[pl.BlockSpec((BB, T, D), lambda b: (b, 0, 0))],
        out_specs=pl.BlockSpec((BB, 1, TOPK), lambda b: (b, 0, 0)),
        out_shape=jax.ShapeDtypeStruct((B, 1, TOPK), jnp.int32),
    )(lite_input)

    idx_flat = indices.reshape(B * TOPK)
    input_rows = input.reshape(B * T, D)

    sc_gather_mean = pl.kernel(
        _sc_gather_mean_body,
        out_type=jax.ShapeDtypeStruct((B, D), jnp.float32),
        mesh=plsc.VectorSubcoreMesh(core_axis_name="c", subcore_axis_name="s"),
        scratch_types=[
            pltpu.VMEM((2 * TOPK,), jnp.int32),
            pltpu.VMEM((2 * TOPK, D), jnp.float32),
            pltpu.VMEM((2, D), jnp.float32),
            pltpu.SemaphoreType.DMA,
        ],
    )
    out = sc_gather_mean(idx_flat, input_rows)

    return out.reshape(B, 1, D)


# speedup vs baseline: 1.0105x; 1.0105x over previous
import functools

import jax
import jax.numpy as jnp
from jax import lax
from jax.experimental import pallas as pl
from jax.experimental.pallas import tpu as pltpu
from jax.experimental.pallas import tpu_sc as plsc

TOPK = 16
BB = 8
NEG_INF = float("-inf")
LANES = 16


def _topk_body(lite_ref, idx_ref, *, T):
    scores = jnp.max(lite_ref[...], axis=2)
    t_iota = jax.lax.broadcasted_iota(jnp.int32, scores.shape, 1)
    b_iota = jax.lax.broadcasted_iota(jnp.int32, scores.shape, 0)
    row_base = (pl.program_id(0) * BB + b_iota) * T
    k_iota = jax.lax.broadcasted_iota(jnp.int32, (BB, TOPK), 1)
    ind_rows = jnp.zeros((BB, TOPK), jnp.int32)
    big = jnp.int32(2**30)
    for k in range(TOPK):
        m = jnp.max(scores, axis=1, keepdims=True)
        cand = jnp.where(scores == m, t_iota, big)
        idx = jnp.min(cand, axis=1, keepdims=True)
        ind_rows = jnp.where(k_iota == k, idx + row_base[:, :1], ind_rows)
        scores = jnp.where(t_iota == idx, NEG_INF, scores)
    l_iota = jax.lax.broadcasted_iota(jnp.int32, (TOPK, BB * TOPK), 1)
    k_col = jax.lax.broadcasted_iota(jnp.int32, (TOPK, BB * TOPK), 0)
    rep_mat = (l_iota % TOPK == k_col).astype(jnp.float32)
    rep = jax.lax.dot(ind_rows.astype(jnp.float32), rep_mat)
    bl_iota = jax.lax.broadcasted_iota(jnp.int32, (BB, BB * TOPK), 1)
    bb_col = jax.lax.broadcasted_iota(jnp.int32, (BB, BB * TOPK), 0)
    keep = bl_iota // TOPK == bb_col
    flat = jnp.sum(jnp.where(keep, rep, 0.0), axis=0, keepdims=True)
    idx_ref[pl.ds(pl.program_id(0), 1), :] = flat.astype(jnp.int32)


def _sc_gather_mean_body(idx_hbm, in_hbm, out_hbm, idx_v, rows_v, out_v, sem):
    nc = 2
    wid = lax.axis_index("s") * nc + lax.axis_index("c")
    rows_per_w = 2 * TOPK
    pltpu.sync_copy(idx_hbm.at[pl.ds(wid * rows_per_w, rows_per_w)], idx_v)
    pltpu.async_copy(in_hbm.at[idx_v], rows_v, sem).wait()
    D = in_hbm.shape[1]
    for bb in range(2):
        def c_body(c, carry):
            sl = pl.ds(c * LANES, LANES)
            acc = rows_v[bb * TOPK, sl]
            for r in range(1, TOPK):
                acc = acc + rows_v[bb * TOPK + r, sl]
            out_v[bb, sl] = acc * (1.0 / TOPK)
            return carry

        lax.fori_loop(0, D // LANES, c_body, 0)
    pltpu.sync_copy(out_v, out_hbm.at[pl.ds(wid * 2, 2)])


@jax.jit
def kernel(input, lite_input):
    B, T, D = input.shape

    indices = pl.pallas_call(
        functools.partial(_topk_body, T=T),
        grid=(B // BB,),
        in_specs=[pl.BlockSpec((BB, T, D), lambda b: (b, 0, 0))],
        out_specs=pl.BlockSpec((B // BB, BB * TOPK), lambda b: (0, 0)),
        out_shape=jax.ShapeDtypeStruct((B // BB, BB * TOPK), jnp.int32),
    )(lite_input)

    idx_flat = indices.reshape(B * TOPK)
    input_rows = input.reshape(B * T, D)

    sc_gather_mean = pl.kernel(
        _sc_gather_mean_body,
        out_type=jax.ShapeDtypeStruct((B, D), jnp.float32),
        mesh=plsc.VectorSubcoreMesh(core_axis_name="c", subcore_axis_name="s"),
        scratch_types=[
            pltpu.VMEM((2 * TOPK,), jnp.int32),
            pltpu.VMEM((2 * TOPK, D), jnp.float32),
            pltpu.VMEM((2, D), jnp.float32),
            pltpu.SemaphoreType.DMA,
        ],
    )
    out = sc_gather_mean(idx_flat, input_rows)

    return out.reshape(B, 1, D)
